# baseline (speedup 1.0000x reference)
import functools

import jax
import jax.numpy as jnp
from jax import lax
from jax.experimental import pallas as pl
from jax.experimental.pallas import tpu as pltpu
from jax.experimental.pallas import tpu_sc as plsc

NC, NS = 2, 16
NW = NC * NS
CHUNK = 512


def _sc_gather(table, idx):
    N = idx.shape[0]
    E = table.shape[1]
    per_w = N // NW
    n_chunks = per_w // CHUNK
    mesh = plsc.VectorSubcoreMesh(core_axis_name="c", subcore_axis_name="s")

    @functools.partial(
        pl.kernel,
        out_type=jax.ShapeDtypeStruct((N, E), table.dtype),
        mesh=mesh,
        compiler_params=pltpu.CompilerParams(use_tc_tiling_on_sc=False),
        scratch_types=[
            pltpu.VMEM((CHUNK,), jnp.int32),
            pltpu.VMEM((CHUNK, E), table.dtype),
            pltpu.SemaphoreType.DMA,
        ],
    )
    def gather_kernel(table_hbm, idx_hbm, out_hbm, idx_v, rows_v, gsem):
        wid = lax.axis_index("s") * NC + lax.axis_index("c")
        base = pl.multiple_of(wid * per_w, CHUNK)

        def body(c, carry):
            off = pl.multiple_of(base + c * CHUNK, CHUNK)
            pltpu.sync_copy(idx_hbm.at[pl.ds(off, CHUNK)], idx_v)
            pltpu.async_copy(table_hbm.at[idx_v], rows_v, gsem).wait()
            pltpu.sync_copy(rows_v, out_hbm.at[pl.ds(off, CHUNK)])
            return carry

        lax.fori_loop(0, n_chunks, body, 0)

    return gather_kernel(table, idx)


def _tc_mlp(h, W1, b1, W2, b2):
    B, K = h.shape
    H = W1.shape[0]
    BM = 128
    grid = (B // BM,)

    def body(hb, w1, b1r, w2, b2r, ob):
        acc = lax.dot_general(hb[...], w1[...], (((1,), (1,)), ((), ())),
                              preferred_element_type=jnp.float32)
        hrelu = jnp.maximum(acc + b1r[...], 0.0)
        s = jnp.sum(hrelu * w2[...], axis=1, keepdims=True)
        ob[...] = s + b2r[0, 0]

    return pl.pallas_call(
        body,
        grid=grid,
        in_specs=[
            pl.BlockSpec((BM, K), lambda i: (i, 0)),
            pl.BlockSpec((H, K), lambda i: (0, 0)),
            pl.BlockSpec((1, H), lambda i: (0, 0)),
            pl.BlockSpec((1, H), lambda i: (0, 0)),
            pl.BlockSpec((1, 1), lambda i: (0, 0)),
        ],
        out_specs=pl.BlockSpec((BM, 1), lambda i: (i, 0)),
        out_shape=jax.ShapeDtypeStruct((B, 1), jnp.float32),
    )(h, W1, b1.reshape(1, H), W2, b2.reshape(1, 1))


def kernel(x, emb, W1, b1, W2, b2):
    B, S = x.shape
    V, E = emb.shape
    idx = x.reshape(-1).astype(jnp.int32)
    emb_bf = emb.astype(jnp.bfloat16)
    emb_i32 = lax.bitcast_convert_type(emb_bf.reshape(V, E // 2, 2), jnp.int32)
    g = _sc_gather(emb_i32, idx)
    h = lax.bitcast_convert_type(g, jnp.bfloat16)
    h = h.reshape(B, S * E)
    W1_bf = W1.astype(jnp.bfloat16)
    return _tc_mlp(h, W1_bf, b1, W2, b2)


# speedup vs baseline: 93.3564x; 93.3564x over previous
import functools

import jax
import jax.numpy as jnp
from jax import lax
from jax.experimental import pallas as pl
from jax.experimental.pallas import tpu as pltpu
from jax.experimental.pallas import tpu_sc as plsc

NC, NS = 2, 16
NW = NC * NS
SC_CHUNK = 50
ROWS = 8 * SC_CHUNK


def _sc_gather_tiled(table, idx_t, B, S):
    V, E = table.shape
    n_chunks = (B // 8) * (S // SC_CHUNK)
    per_w = n_chunks // NW
    CB = SC_CHUNK * E
    mesh = plsc.VectorSubcoreMesh(core_axis_name="c", subcore_axis_name="s")

    @functools.partial(
        pl.kernel,
        out_type=jax.ShapeDtypeStruct((B, S * E), table.dtype),
        mesh=mesh,
        scratch_types=[
            pltpu.VMEM((ROWS,), jnp.int32),
            pltpu.VMEM((ROWS, E), table.dtype),
            pltpu.SemaphoreType.DMA,
        ],
    )
    def gather_kernel(table_hbm, idx_hbm, out_hbm, idx_v, rows_v, gsem):
        wid = lax.axis_index("s") * NC + lax.axis_index("c")
        g0 = wid * per_w

        def body(i, carry):
            g = g0 + i
            bt = g // (S // SC_CHUNK)
            sc = lax.rem(g, S // SC_CHUNK)
            off = pl.multiple_of(g * ROWS, ROWS)
            pltpu.sync_copy(idx_hbm.at[pl.ds(off, ROWS)], idx_v)
            pltpu.async_copy(table_hbm.at[idx_v], rows_v, gsem).wait()
            pltpu.sync_copy(
                rows_v.reshape(8, CB),
                out_hbm.at[pl.ds(bt * 8, 8), pl.ds(sc * CB, CB)],
            )
            return carry

        lax.fori_loop(0, per_w, body, 0)

    return gather_kernel(table, idx_t)


def _tc_mlp(h, W1, b1, W2, b2):
    B, K = h.shape
    H = W1.shape[0]
    BM = 128
    grid = (B // BM,)

    def body(hb, w1, b1r, w2, b2r, ob):
        acc = lax.dot_general(hb[...], w1[...], (((1,), (1,)), ((), ())),
                              preferred_element_type=jnp.float32)
        hrelu = jnp.maximum(acc + b1r[...], 0.0)
        s = jnp.sum(hrelu * w2[...], axis=1, keepdims=True)
        ob[...] = s + b2r[0, 0]

    return pl.pallas_call(
        body,
        grid=grid,
        in_specs=[
            pl.BlockSpec((BM, K), lambda i: (i, 0)),
            pl.BlockSpec((H, K), lambda i: (0, 0)),
            pl.BlockSpec((1, H), lambda i: (0, 0)),
            pl.BlockSpec((1, H), lambda i: (0, 0)),
            pl.BlockSpec((1, 1), lambda i: (0, 0)),
        ],
        out_specs=pl.BlockSpec((BM, 1), lambda i: (i, 0)),
        out_shape=jax.ShapeDtypeStruct((B, 1), jnp.float32),
    )(h, W1, b1.reshape(1, H), W2, b2.reshape(1, 1))


def kernel(x, emb, W1, b1, W2, b2):
    B, S = x.shape
    V, E = emb.shape
    idx_t = (x.astype(jnp.int32)
             .reshape(B // 8, 8, S // SC_CHUNK, SC_CHUNK)
             .transpose(0, 2, 1, 3)
             .reshape(-1))
    h = _sc_gather_tiled(emb, idx_t, B, S)
    return _tc_mlp(h, W1, b1, W2, b2)


# speedup vs baseline: 104.8143x; 1.1227x over previous
import functools

import jax
import jax.numpy as jnp
from jax import lax
from jax.experimental import pallas as pl
from jax.experimental.pallas import tpu as pltpu
from jax.experimental.pallas import tpu_sc as plsc

NC, NS = 2, 16
NW = NC * NS
SC_CHUNK = 40
ROWS = 8 * SC_CHUNK


def _sc_gather_tiled(table, idx_t, B, S):
    V, E = table.shape
    SPB = S // SC_CHUNK
    n_chunks = (B // 8) * SPB
    per_w = n_chunks // NW
    CB = SC_CHUNK * E
    mesh = plsc.VectorSubcoreMesh(core_axis_name="c", subcore_axis_name="s")

    @functools.partial(
        pl.kernel,
        out_type=jax.ShapeDtypeStruct((B, S * E), table.dtype),
        mesh=mesh,
        scratch_types=[
            pltpu.VMEM((per_w * ROWS,), jnp.int32),
            pltpu.VMEM((2, ROWS, E), table.dtype),
            pltpu.SemaphoreType.DMA,
            pltpu.SemaphoreType.DMA,
            pltpu.SemaphoreType.DMA,
            pltpu.SemaphoreType.DMA,
        ],
    )
    def gather_kernel(table_hbm, idx_hbm, out_hbm, idx_all, rows_v,
                      gsem0, gsem1, wsem0, wsem1):
        wid = lax.axis_index("s") * NC + lax.axis_index("c")
        g0 = wid * per_w
        gsems = (gsem0, gsem1)
        wsems = (wsem0, wsem1)

        pltpu.sync_copy(idx_hbm.at[pl.ds(pl.multiple_of(g0 * ROWS, ROWS),
                                         per_w * ROWS)], idx_all)

        def idx_slice(i):
            return idx_all.at[pl.ds(pl.multiple_of(i * ROWS, ROWS), ROWS)]

        def dst(g):
            bt = g // SPB
            sc = lax.rem(g, SPB)
            return out_hbm.at[pl.ds(bt * 8, 8), pl.ds(sc * CB, CB)]

        def start_gather(i, slot):
            pltpu.async_copy(table_hbm.at[idx_slice(i)], rows_v.at[slot],
                             gsems[slot])

        def wait_gather(i, slot):
            pltpu.make_async_copy(table_hbm.at[idx_slice(i)],
                                  rows_v.at[slot], gsems[slot]).wait()

        def start_write(g, slot):
            pltpu.async_copy(rows_v.at[slot].reshape(8, CB), dst(g),
                             wsems[slot])

        def wait_write(g, slot):
            pltpu.make_async_copy(rows_v.at[slot].reshape(8, CB), dst(g),
                                  wsems[slot]).wait()

        def do_iter(i, cur, nxt):
            @pl.when(i + 1 < per_w)
            def _():
                @pl.when(i >= 1)
                def _():
                    wait_write(g0 + i - 1, nxt)
                start_gather(i + 1, nxt)

            wait_gather(i, cur)
            start_write(g0 + i, cur)

        start_gather(0, 0)

        def body(i, carry):
            @pl.when(lax.rem(i, 2) == 0)
            def _():
                do_iter(i, 0, 1)

            @pl.when(lax.rem(i, 2) == 1)
            def _():
                do_iter(i, 1, 0)
            return carry

        lax.fori_loop(0, per_w, body, 0)
        wait_write(g0 + per_w - 2, (per_w - 2) % 2)
        wait_write(g0 + per_w - 1, (per_w - 1) % 2)

    return gather_kernel(table, idx_t)


def _tc_mlp(h, W1, b1, W2, b2):
    B, K = h.shape
    H = W1.shape[0]
    BM = 128
    grid = (B // BM,)

    def body(hb, w1, b1r, w2, b2r, ob):
        acc = lax.dot_general(hb[...], w1[...], (((1,), (1,)), ((), ())),
                              preferred_element_type=jnp.float32)
        hrelu = jnp.maximum(acc + b1r[...], 0.0)
        s = jnp.sum(hrelu * w2[...], axis=1, keepdims=True)
        ob[...] = s + b2r[0, 0]

    return pl.pallas_call(
        body,
        grid=grid,
        in_specs=[
            pl.BlockSpec((BM, K), lambda i: (i, 0)),
            pl.BlockSpec((H, K), lambda i: (0, 0)),
            pl.BlockSpec((1, H), lambda i: (0, 0)),
            pl.BlockSpec((1, H), lambda i: (0, 0)),
            pl.BlockSpec((1, 1), lambda i: (0, 0)),
        ],
        out_specs=pl.BlockSpec((BM, 1), lambda i: (i, 0)),
        out_shape=jax.ShapeDtypeStruct((B, 1), jnp.float32),
    )(h, W1, b1.reshape(1, H), W2, b2.reshape(1, 1))


def kernel(x, emb, W1, b1, W2, b2):
    B, S = x.shape
    V, E = emb.shape
    idx_t = (x.astype(jnp.int32)
             .reshape(B // 8, 8, S // SC_CHUNK, SC_CHUNK)
             .transpose(0, 2, 1, 3)
             .reshape(-1))
    h = _sc_gather_tiled(emb, idx_t, B, S)
    return _tc_mlp(h, W1, b1, W2, b2)


# speedup vs baseline: 107.0956x; 1.0218x over previous
import functools

import jax
import jax.numpy as jnp
from jax import lax
from jax.experimental import pallas as pl
from jax.experimental.pallas import tpu as pltpu
from jax.experimental.pallas import tpu_sc as plsc

NC, NS = 2, 16
NW = NC * NS
SC_CHUNK = 40
ROWS = 8 * SC_CHUNK


def _sc_gather_tiled(table, idx_t, B, S):
    V, E = table.shape
    SPB = S // SC_CHUNK
    n_chunks = (B // 8) * SPB
    per_w = n_chunks // NW
    CB = SC_CHUNK * E
    mesh = plsc.VectorSubcoreMesh(core_axis_name="c", subcore_axis_name="s")

    @functools.partial(
        pl.kernel,
        out_type=jax.ShapeDtypeStruct((B, S * E), table.dtype),
        mesh=mesh,
        scratch_types=[
            pltpu.VMEM((per_w * ROWS,), jnp.int32),
            pltpu.VMEM((2, ROWS, E), table.dtype),
            pltpu.SemaphoreType.DMA,
            pltpu.SemaphoreType.DMA,
            pltpu.SemaphoreType.DMA,
            pltpu.SemaphoreType.DMA,
        ],
    )
    def gather_kernel(table_hbm, idx_hbm, out_hbm, idx_all, rows_v,
                      gsem0, gsem1, wsem0, wsem1):
        wid = lax.axis_index("s") * NC + lax.axis_index("c")
        g0 = wid * per_w
        gsems = (gsem0, gsem1)
        wsems = (wsem0, wsem1)

        pltpu.sync_copy(idx_hbm.at[pl.ds(pl.multiple_of(g0 * ROWS, ROWS),
                                         per_w * ROWS)], idx_all)

        def idx_slice(i):
            return idx_all.at[pl.ds(pl.multiple_of(i * ROWS, ROWS), ROWS)]

        def dst(g):
            bt = g // SPB
            sc = lax.rem(g, SPB)
            return out_hbm.at[pl.ds(bt * 8, 8), pl.ds(sc * CB, CB)]

        def start_gather(i, slot):
            pltpu.async_copy(table_hbm.at[idx_slice(i)], rows_v.at[slot],
                             gsems[slot])

        def wait_gather(i, slot):
            pltpu.make_async_copy(table_hbm.at[idx_slice(i)],
                                  rows_v.at[slot], gsems[slot]).wait()

        def start_write(g, slot):
            pltpu.async_copy(rows_v.at[slot].reshape(8, CB), dst(g),
                             wsems[slot])

        def wait_write(g, slot):
            pltpu.make_async_copy(rows_v.at[slot].reshape(8, CB), dst(g),
                                  wsems[slot]).wait()

        def do_iter(i, cur, nxt):
            @pl.when(i + 1 < per_w)
            def _():
                @pl.when(i >= 1)
                def _():
                    wait_write(g0 + i - 1, nxt)
                start_gather(i + 1, nxt)

            wait_gather(i, cur)
            start_write(g0 + i, cur)

        start_gather(0, 0)

        def body(i, carry):
            @pl.when(lax.rem(i, 2) == 0)
            def _():
                do_iter(i, 0, 1)

            @pl.when(lax.rem(i, 2) == 1)
            def _():
                do_iter(i, 1, 0)
            return carry

        lax.fori_loop(0, per_w, body, 0)
        wait_write(g0 + per_w - 2, (per_w - 2) % 2)
        wait_write(g0 + per_w - 1, (per_w - 1) % 2)

    return gather_kernel(table, idx_t)


def _tc_mlp(h, W1, b1, W2, b2):
    B, K = h.shape
    H = W1.shape[0]
    BM = 128
    grid = (B // BM,)

    def body(hb, w1, b1r, w2, b2r, ob):
        acc = lax.dot_general(hb[...], w1[...], (((1,), (1,)), ((), ())),
                              preferred_element_type=jnp.float32)
        hrelu = jnp.maximum(acc + b1r[...], 0.0)
        s = jnp.sum(hrelu * w2[...], axis=1, keepdims=True)
        ob[...] = s + b2r[0, 0]

    return pl.pallas_call(
        body,
        grid=grid,
        in_specs=[
            pl.BlockSpec((BM, K), lambda i: (i, 0)),
            pl.BlockSpec((H, K), lambda i: (0, 0)),
            pl.BlockSpec((1, H), lambda i: (0, 0)),
            pl.BlockSpec((1, H), lambda i: (0, 0)),
            pl.BlockSpec((1, 1), lambda i: (0, 0)),
        ],
        out_specs=pl.BlockSpec((BM, 1), lambda i: (i, 0)),
        out_shape=jax.ShapeDtypeStruct((B, 1), jnp.float32),
    )(h, W1, b1.reshape(1, H), W2, b2.reshape(1, 1))


def kernel(x, emb, W1, b1, W2, b2):
    B, S = x.shape
    V, E = emb.shape
    BH = B // 2
    outs = []
    for p in range(2):
        xp = lax.slice_in_dim(x, p * BH, (p + 1) * BH, axis=0)
        idx_t = (xp.astype(jnp.int32)
                 .reshape(BH // 8, 8, S // SC_CHUNK, SC_CHUNK)
                 .transpose(0, 2, 1, 3)
                 .reshape(-1))
        h = _sc_gather_tiled(emb, idx_t, BH, S)
        outs.append(_tc_mlp(h, W1, b1, W2, b2))
    return jnp.concatenate(outs, axis=0)
